# baseline (speedup 1.0000x reference)
import jax
import jax.numpy as jnp
from jax import lax
from jax.experimental import pallas as pl
from jax.experimental.pallas import tpu as pltpu
from jax.experimental.pallas import tpu_sc as plsc

P = 48000
C = 64
B = 4
NY = 496
NX = 432
CANVAS = NY * NX
S_TOT = B * CANVAS

NC = 2
NS = 16
NW = NC * NS
SLOTS_PER_TILE = S_TOT // NW
ROWS_PER_TILE = SLOTS_PER_TILE // NX
TILES_PER_BATCH = NY // ROWS_PER_TILE

KEY_CHUNK = 6000
LISTCAP = 448
TILE_CAP = 35072
FB = NW * TILE_CAP // 128 + 17
META_LEN = NW * 64
NBUF = 24


def _sc_body(vf2, keys, feat, slot_c, starts, cnts,
             map_v, keysbuf, ring, pid_buf, slot_buf, meta_s, meta_c,
             sem_g, sem_f, sem_s):
    wid = lax.axis_index("c") * NS + lax.axis_index("s")
    tile_base = wid * SLOTS_PER_TILE
    ent_base = wid * TILE_CAP

    iota = lax.iota(jnp.int32, 16)
    zi = jnp.zeros((16,), jnp.int32)
    neg1 = jnp.full((16,), -1, jnp.int32)
    lane0 = iota == 0

    def init_map(i, carry):
        map_v[pl.ds(i * 16, 16)] = neg1
        return carry
    lax.fori_loop(0, SLOTS_PER_TILE // 16, init_map, 0)

    def init_lists(i, carry):
        pid_buf[pl.ds(i * 16, 16)] = zi
        slot_buf[0, pl.ds(i * 16, 16)] = zi
        slot_buf[1, pl.ds(i * 16, 16)] = zi
        return carry
    lax.fori_loop(0, LISTCAP // 16, init_lists, 0)

    def chunk_body(ci, carry):
        base_p = ci * KEY_CHUNK
        pltpu.sync_copy(keys.at[pl.ds(base_p, KEY_CHUNK)], keysbuf)

        def vec_body(i, carry2):
            k = keysbuf[pl.ds(i * 16, 16)]
            rel = k - tile_base
            m = (rel >= 0) & (rel < SLOTS_PER_TILE)
            relc = jnp.clip(rel, 0, SLOTS_PER_TILE - 1)
            pid = base_p + i * 16 + iota
            plsc.store_scatter(map_v, [relc], pid, mask=m)
            return carry2
        return lax.fori_loop(0, KEY_CHUNK // 16, vec_body, carry)
    lax.fori_loop(0, P // KEY_CHUNK, chunk_body, 0)

    def drain_feat(n, carry):
        def d(i, c2):
            pltpu.make_async_copy(feat.at[0, pl.ds(0, 16), :], ring.at[0],
                                  sem_f).wait()
            return c2
        return lax.fori_loop(0, n, d, carry)

    def drain_slot(n):
        def d(i, c2):
            pltpu.make_async_copy(slot_c.at[0, pl.ds(0, 16)],
                                  slot_buf.at[0, pl.ds(0, 16)], sem_s).wait()
            return c2
        lax.fori_loop(0, n, d, 0)

    def row_body(r, carry):
        off, rp, ns0, ns1 = carry
        par = r % 2
        pns = jnp.where(par == 0, ns0, ns1)
        drain_slot(pns)

        row_off = r * NX

        def comp_body(j, k):
            m16 = map_v[pl.ds(row_off + j * 16, 16)]
            msk = m16 >= 0
            plsc.store_compressed(pid_buf.at[pl.ds(k, 16)],
                                  jnp.right_shift(m16, 1), mask=msk)
            sv = (j * 16 + iota) | jnp.left_shift(m16 & 1, 9)
            plsc.store_compressed(slot_buf.at[par, pl.ds(k, 16)], sv,
                                  mask=msk)
            cnt = plsc.all_reduce_population_count(msk)
            return k + cnt[0]
        kt = lax.fori_loop(0, NX // 16, comp_body, 0)
        nch = (kt + 15) // 16

        plsc.store_scatter(meta_s, [jnp.full((16,), r, jnp.int32)],
                           jnp.full((16,), ent_base + off, jnp.int32),
                           mask=lane0)
        plsc.store_scatter(meta_c, [jnp.full((16,), r, jnp.int32)],
                           jnp.full((16,), kt, jnp.int32), mask=lane0)

        nseg = (nch + 11) // 12

        def seg_body(s, carry2):
            g0 = s * 12
            gn = jnp.minimum(nch - g0, 12)

            def g_body(gg, c3):
                g = g0 + gg
                slot = (rp + g) % NBUF

                @pl.when(rp + g >= NBUF)
                def _():
                    drain_feat(1, 0)
                pltpu.async_copy(vf2.at[pid_buf.at[pl.ds(g * 16, 16)]],
                                 ring.at[slot], sem_g)
                return c3
            lax.fori_loop(0, gn, g_body, 0)

            def e_body(gg, c3):
                g = g0 + gg
                slot = (rp + g) % NBUF
                pltpu.make_async_copy(vf2.at[pid_buf.at[pl.ds(g * 16, 16)]],
                                      ring.at[slot], sem_g).wait()
                eoff = ent_base + off + g * 16
                eb = eoff // 128
                er = pl.multiple_of(eoff % 128, 8)
                pltpu.async_copy(ring.at[slot],
                                 feat.at[eb, pl.ds(er, 16), :], sem_f)
                pltpu.async_copy(slot_buf.at[par, pl.ds(g * 16, 16)],
                                 slot_c.at[eb, pl.ds(er, 16)], sem_s)
                return c3
            lax.fori_loop(0, gn, e_body, 0)
            return carry2
        lax.fori_loop(0, nseg, seg_body, 0)

        ns0n = jnp.where(par == 0, nch, ns0)
        ns1n = jnp.where(par == 1, nch, ns1)
        return (off + ((kt + 127) // 128) * 128, rp + nch, ns0n, ns1n)

    off, rp, ns0, ns1 = lax.fori_loop(0, ROWS_PER_TILE, row_body,
                                      (0, 0, 0, 0))
    drain_feat(jnp.minimum(rp, NBUF), 0)
    drain_slot(ns0)
    drain_slot(ns1)

    moff = pl.multiple_of(wid * 64, 8)
    pltpu.sync_copy(meta_s, starts.at[pl.ds(moff, 64)])
    pltpu.sync_copy(meta_c, cnts.at[pl.ds(moff, 64)])


def _sc_stage(vf2, keys):
    f = pl.kernel(
        _sc_body,
        out_type=(
            jax.ShapeDtypeStruct((FB, 128, 128), jnp.float32),
            jax.ShapeDtypeStruct((FB, 128), jnp.int32),
            jax.ShapeDtypeStruct((META_LEN,), jnp.int32),
            jax.ShapeDtypeStruct((META_LEN,), jnp.int32),
        ),
        mesh=plsc.VectorSubcoreMesh(core_axis_name="c", subcore_axis_name="s"),
        compiler_params=pltpu.CompilerParams(needs_layout_passes=False,
                                             use_tc_tiling_on_sc=False),
        scratch_types=[
            pltpu.VMEM((SLOTS_PER_TILE,), jnp.int32),
            pltpu.VMEM((KEY_CHUNK,), jnp.int32),
            pltpu.VMEM((NBUF, 16, 128), jnp.float32),
            pltpu.VMEM((LISTCAP,), jnp.int32),
            pltpu.VMEM((2, LISTCAP), jnp.int32),
            pltpu.VMEM((64,), jnp.int32),
            pltpu.VMEM((64,), jnp.int32),
            pltpu.SemaphoreType.DMA,
            pltpu.SemaphoreType.DMA,
            pltpu.SemaphoreType.DMA,
        ],
    )
    return f(vf2, keys)


RB = 16


def _tc_body(starts_sm, cnts_sm, midx_sm, feat, slot_c, o_ref,
             fbufA, fbufB, sloA, sloB, fbufX, xbuf, xslo,
             sems_ab, sems_x, sem_x1):
    bb = pl.program_id(0)
    yt = pl.program_id(1)
    iota_t = lax.broadcasted_iota(jnp.int32, (32, NX), 1)
    iota_t512 = iota_t + 512
    iota_t128 = lax.broadcasted_iota(jnp.int32, (128, NX), 1)
    iota_t128_512 = iota_t128 + 512
    dn = (((0,), (0,)), ((), ()))

    NYT = NY // RB
    NSTEP = B * NYT
    si = bb * NYT + yt
    par = si % 2

    def metas_for(s):
        sb = s // NYT
        syt = s - sb * NYT
        g0 = sb * NY + syt * RB
        out = []
        for rr in range(RB):
            midx = midx_sm[g0 + rr]
            out.append((pl.multiple_of(starts_sm[midx], 128), cnts_sm[midx]))
        return out

    def blk_cps(p, bA, bB):
        return [
            pltpu.make_async_copy(feat.at[pl.ds(bA, RB), pl.ds(0, 32), :],
                                  fbufA.at[p], sems_ab.at[p, 0]),
            pltpu.make_async_copy(feat.at[pl.ds(bB, RB), pl.ds(0, 32), :],
                                  fbufB.at[p], sems_ab.at[p, 1]),
            pltpu.make_async_copy(slot_c.at[pl.ds(bA, RB), :],
                                  sloA.at[p], sems_ab.at[p, 2]),
            pltpu.make_async_copy(slot_c.at[pl.ds(bB, RB), :],
                                  sloB.at[p], sems_ab.at[p, 3]),
        ]

    def xchunk_cp(p, rr, blk, cc):
        return pltpu.make_async_copy(
            feat.at[blk, pl.ds(32 * cc, 32), :],
            fbufX.at[p, rr, cc - 1], sems_x.at[p, rr, cc - 1])

    def fire_all(s, p):
        ms = metas_for(s)
        bA = ms[0][0] // 128
        bB = ms[RB - 1][0] // 128 - (RB - 1)
        for cp in blk_cps(p, bA, bB):
            cp.start()
        for rr in range(RB):
            start, cnt = ms[rr]
            fast = jnp.logical_and(
                jnp.logical_or(start == (bA + rr) * 128,
                               start == (bB + rr) * 128), cnt <= 128)
            for cc in range(1, 4):
                @pl.when(jnp.logical_and(fast, cnt > 32 * cc))
                def _(p=p, rr=rr, start=start, cc=cc):
                    xchunk_cp(p, rr, start // 128, cc).start()

    def chunk_acc(f, sm_t, cc):
        sub = lax.slice(sm_t, (32 * cc, 0), (32 * cc + 32, 1))
        oh_lo = (sub == iota_t).astype(jnp.bfloat16)
        oh_hi = (sub == iota_t512).astype(jnp.bfloat16)
        dlo = lax.dot_general(f[:, :C], oh_lo, dn,
                              preferred_element_type=jnp.float32)
        dhi = lax.dot_general(f[:, C:], oh_hi, dn,
                              preferred_element_type=jnp.float32)
        return dlo + dhi

    @pl.when(si == 0)
    def _():
        fire_all(si, par)

    @pl.when(si + 1 < NSTEP)
    def _():
        fire_all(si + 1, 1 - par)

    metas = metas_for(si)
    bA = metas[0][0] // 128
    bB = metas[RB - 1][0] // 128 - (RB - 1)
    for cp in blk_cps(par, bA, bB):
        cp.wait()

    for rr in range(RB):
        start, cnt = metas[rr]
        inA = start == (bA + rr) * 128
        inB = start == (bB + rr) * 128
        fast = jnp.logical_and(jnp.logical_or(inA, inB), cnt <= 128)

        @pl.when(fast)
        def _(rr=rr, start=start, cnt=cnt, inA=inA):
            svec = jnp.where(inA, sloA[par, rr], sloB[par, rr])
            sm = jnp.where(lax.iota(jnp.int32, 128) < cnt, svec, 4096)
            sm_t = jnp.transpose(sm.reshape(1, 128), (1, 0))
            f0 = jnp.where(inA, fbufA[par, rr], fbufB[par, rr])
            o_ref[0, :, rr, :] = chunk_acc(f0, sm_t, 0)
            for cc in range(1, 4):
                @pl.when(cnt > 32 * cc)
                def _(cc=cc, rr=rr, sm_t=sm_t):
                    xchunk_cp(par, rr, start // 128, cc).wait()
                    o_ref[0, :, rr, :] += chunk_acc(fbufX[par, rr, cc - 1],
                                                    sm_t, cc)

        @pl.when(jnp.logical_not(fast))
        def _(rr=rr, start=start, cnt=cnt):
            o_ref[0, :, rr, :] = jnp.zeros((C, NX), jnp.float32)

            def big(c, carry):
                pltpu.make_async_copy(feat.at[start // 128 + c], xbuf,
                                      sem_x1).start()
                pltpu.make_async_copy(feat.at[start // 128 + c], xbuf,
                                      sem_x1).wait()
                pltpu.make_async_copy(slot_c.at[start // 128 + c], xslo,
                                      sem_x1).start()
                pltpu.make_async_copy(slot_c.at[start // 128 + c], xslo,
                                      sem_x1).wait()
                f = xbuf[...]
                ent = lax.iota(jnp.int32, 128) + 128 * c
                smx = jnp.where(ent < cnt, xslo[...], 4096)
                smx_t = jnp.transpose(smx.reshape(1, 128), (1, 0))
                oh_lo = (smx_t == iota_t128).astype(jnp.bfloat16)
                oh_hi = (smx_t == iota_t128_512).astype(jnp.bfloat16)
                o_ref[0, :, rr, :] += (
                    lax.dot_general(f[:, :C], oh_lo, dn,
                                    preferred_element_type=jnp.float32)
                    + lax.dot_general(f[:, C:], oh_hi, dn,
                                      preferred_element_type=jnp.float32))
                return carry
            lax.fori_loop(0, (cnt + 127) // 128, big, 0)


def _tc_stage(feat, slot_c, starts, cnts):
    grid_spec = pltpu.PrefetchScalarGridSpec(
        num_scalar_prefetch=3,
        grid=(B, NY // RB),
        in_specs=[
            pl.BlockSpec(memory_space=pltpu.MemorySpace.HBM),
            pl.BlockSpec(memory_space=pltpu.MemorySpace.HBM),
        ],
        out_specs=pl.BlockSpec((1, C, RB, NX),
                               lambda b, y, s_r, c_r, m_r: (b, 0, y, 0)),
        scratch_shapes=[
            pltpu.VMEM((2, RB, 32, 128), jnp.float32),
            pltpu.VMEM((2, RB, 32, 128), jnp.float32),
            pltpu.VMEM((2, RB, 128), jnp.int32),
            pltpu.VMEM((2, RB, 128), jnp.int32),
            pltpu.VMEM((2, RB, 3, 32, 128), jnp.float32),
            pltpu.VMEM((128, 128), jnp.float32),
            pltpu.VMEM((128,), jnp.int32),
            pltpu.SemaphoreType.DMA((2, 4)),
            pltpu.SemaphoreType.DMA((2, RB, 3)),
            pltpu.SemaphoreType.DMA,
        ],
    )
    midx_map = ((jnp.arange(B * NY, dtype=jnp.int32) // ROWS_PER_TILE) * 64
                + jnp.arange(B * NY, dtype=jnp.int32) % ROWS_PER_TILE)
    return pl.pallas_call(
        _tc_body,
        grid_spec=grid_spec,
        out_shape=jax.ShapeDtypeStruct((B, C, NY, NX), jnp.float32),
    )(starts, cnts, midx_map, feat, slot_c)


def kernel(voxel_features, coords, batch_size, output_shape):
    c0 = coords[:, 0]
    key = c0 * CANVAS + coords[:, 2] * NX + coords[:, 3]
    key = jnp.where(c0 < batch_size, key, S_TOT).astype(jnp.int32)
    vf2 = voxel_features.reshape(P // 2, 2 * C)
    feat, slot_c, starts, cnts = _sc_stage(vf2, key)
    return _tc_stage(feat, slot_c, starts, cnts)


# speedup vs baseline: 1.0262x; 1.0262x over previous
import jax
import jax.numpy as jnp
from jax import lax
from jax.experimental import pallas as pl
from jax.experimental.pallas import tpu as pltpu
from jax.experimental.pallas import tpu_sc as plsc

P = 48000
C = 64
B = 4
NY = 496
NX = 432
CANVAS = NY * NX
S_TOT = B * CANVAS

NC = 2
NS = 16
NW = NC * NS
SLOTS_PER_TILE = S_TOT // NW
ROWS_PER_TILE = SLOTS_PER_TILE // NX
TILES_PER_BATCH = NY // ROWS_PER_TILE

KEY_CHUNK = 6000
LISTCAP = 448
TILE_CAP = 35072
FB = NW * TILE_CAP // 128 + 17
META_LEN = NW * 64
NBUF = 24


def _sc_body(vf2, keys, feat, slot_c, starts, cnts,
             map_v, keysbuf, ring, pid_buf, slot_buf, meta_s, meta_c,
             sem_g, sem_f, sem_s):
    wid = lax.axis_index("c") * NS + lax.axis_index("s")
    tile_base = wid * SLOTS_PER_TILE
    ent_base = wid * TILE_CAP

    iota = lax.iota(jnp.int32, 16)
    zi = jnp.zeros((16,), jnp.int32)
    neg1 = jnp.full((16,), -1, jnp.int32)
    lane0 = iota == 0

    def init_map(i, carry):
        map_v[pl.ds(i * 16, 16)] = neg1
        return carry
    lax.fori_loop(0, SLOTS_PER_TILE // 16, init_map, 0)

    def init_lists(i, carry):
        pid_buf[pl.ds(i * 16, 16)] = zi
        slot_buf[0, pl.ds(i * 16, 16)] = zi
        slot_buf[1, pl.ds(i * 16, 16)] = zi
        return carry
    lax.fori_loop(0, LISTCAP // 16, init_lists, 0)

    def chunk_body(ci, carry):
        base_p = ci * KEY_CHUNK
        pltpu.sync_copy(keys.at[pl.ds(base_p, KEY_CHUNK)], keysbuf)

        def vec_body(i, carry2):
            k = keysbuf[pl.ds(i * 16, 16)]
            rel = k - tile_base
            m = (rel >= 0) & (rel < SLOTS_PER_TILE)
            relc = jnp.clip(rel, 0, SLOTS_PER_TILE - 1)
            pid = base_p + i * 16 + iota
            plsc.store_scatter(map_v, [relc], pid, mask=m)
            return carry2
        return lax.fori_loop(0, KEY_CHUNK // 16, vec_body, carry)
    lax.fori_loop(0, P // KEY_CHUNK, chunk_body, 0)

    def drain_feat(n, carry):
        def d(i, c2):
            pltpu.make_async_copy(feat.at[0, pl.ds(0, 16), :], ring.at[0],
                                  sem_f).wait()
            return c2
        return lax.fori_loop(0, n, d, carry)

    def drain_slot(n):
        def d(i, c2):
            pltpu.make_async_copy(slot_c.at[0, pl.ds(0, 16)],
                                  slot_buf.at[0, pl.ds(0, 16)], sem_s).wait()
            return c2
        lax.fori_loop(0, n, d, 0)

    def row_body(r, carry):
        off, rp, ns0, ns1 = carry
        par = r % 2
        pns = jnp.where(par == 0, ns0, ns1)
        drain_slot(pns)

        row_off = r * NX

        def comp_body(j, k):
            m16 = map_v[pl.ds(row_off + j * 16, 16)]
            msk = m16 >= 0
            plsc.store_compressed(pid_buf.at[pl.ds(k, 16)],
                                  jnp.right_shift(m16, 1), mask=msk)
            sv = (j * 16 + iota) | jnp.left_shift(m16 & 1, 9)
            plsc.store_compressed(slot_buf.at[par, pl.ds(k, 16)], sv,
                                  mask=msk)
            cnt = plsc.all_reduce_population_count(msk)
            return k + cnt[0]
        kt = lax.fori_loop(0, NX // 16, comp_body, 0)
        nch = (kt + 15) // 16

        plsc.store_scatter(meta_s, [jnp.full((16,), r, jnp.int32)],
                           jnp.full((16,), ent_base + off, jnp.int32),
                           mask=lane0)
        plsc.store_scatter(meta_c, [jnp.full((16,), r, jnp.int32)],
                           jnp.full((16,), kt, jnp.int32), mask=lane0)

        nseg = (nch + 11) // 12

        def seg_body(s, carry2):
            g0 = s * 12
            gn = jnp.minimum(nch - g0, 12)

            def g_body(gg, c3):
                g = g0 + gg
                slot = (rp + g) % NBUF

                @pl.when(rp + g >= NBUF)
                def _():
                    drain_feat(1, 0)
                pltpu.async_copy(vf2.at[pid_buf.at[pl.ds(g * 16, 16)]],
                                 ring.at[slot], sem_g)
                return c3
            lax.fori_loop(0, gn, g_body, 0)

            def e_body(gg, c3):
                g = g0 + gg
                slot = (rp + g) % NBUF
                pltpu.make_async_copy(vf2.at[pid_buf.at[pl.ds(g * 16, 16)]],
                                      ring.at[slot], sem_g).wait()
                half_v = jnp.right_shift(
                    slot_buf[par, pl.ds(g * 16, 16)], 9) * 64
                for fw in range(C):
                    vals = plsc.load_gather(ring.at[slot],
                                            [iota, half_v + fw])
                    plsc.store_scatter(ring.at[slot],
                                       [iota, jnp.full((16,), fw, jnp.int32)],
                                       vals)
                eoff = ent_base + off + g * 16
                eb = eoff // 128
                er = pl.multiple_of(eoff % 128, 8)
                pltpu.async_copy(ring.at[slot],
                                 feat.at[eb, pl.ds(er, 16), :], sem_f)
                pltpu.async_copy(slot_buf.at[par, pl.ds(g * 16, 16)],
                                 slot_c.at[eb, pl.ds(er, 16)], sem_s)
                return c3
            lax.fori_loop(0, gn, e_body, 0)
            return carry2
        lax.fori_loop(0, nseg, seg_body, 0)

        ns0n = jnp.where(par == 0, nch, ns0)
        ns1n = jnp.where(par == 1, nch, ns1)
        return (off + ((kt + 127) // 128) * 128, rp + nch, ns0n, ns1n)

    off, rp, ns0, ns1 = lax.fori_loop(0, ROWS_PER_TILE, row_body,
                                      (0, 0, 0, 0))
    drain_feat(jnp.minimum(rp, NBUF), 0)
    drain_slot(ns0)
    drain_slot(ns1)

    moff = pl.multiple_of(wid * 64, 8)
    pltpu.sync_copy(meta_s, starts.at[pl.ds(moff, 64)])
    pltpu.sync_copy(meta_c, cnts.at[pl.ds(moff, 64)])


def _sc_stage(vf2, keys):
    f = pl.kernel(
        _sc_body,
        out_type=(
            jax.ShapeDtypeStruct((FB, 128, 128), jnp.float32),
            jax.ShapeDtypeStruct((FB, 128), jnp.int32),
            jax.ShapeDtypeStruct((META_LEN,), jnp.int32),
            jax.ShapeDtypeStruct((META_LEN,), jnp.int32),
        ),
        mesh=plsc.VectorSubcoreMesh(core_axis_name="c", subcore_axis_name="s"),
        compiler_params=pltpu.CompilerParams(needs_layout_passes=False,
                                             use_tc_tiling_on_sc=False),
        scratch_types=[
            pltpu.VMEM((SLOTS_PER_TILE,), jnp.int32),
            pltpu.VMEM((KEY_CHUNK,), jnp.int32),
            pltpu.VMEM((NBUF, 16, 128), jnp.float32),
            pltpu.VMEM((LISTCAP,), jnp.int32),
            pltpu.VMEM((2, LISTCAP), jnp.int32),
            pltpu.VMEM((64,), jnp.int32),
            pltpu.VMEM((64,), jnp.int32),
            pltpu.SemaphoreType.DMA,
            pltpu.SemaphoreType.DMA,
            pltpu.SemaphoreType.DMA,
        ],
    )
    return f(vf2, keys)


RB = 16


def _tc_body(starts_sm, cnts_sm, midx_sm, feat, slot_c, o_ref,
             fbufA, fbufB, sloA, sloB, fbufX, xbuf, xslo,
             sems_ab, sems_x, sem_x1):
    bb = pl.program_id(0)
    yt = pl.program_id(1)
    iota_t = lax.broadcasted_iota(jnp.int32, (32, NX), 1)
    iota_t128 = lax.broadcasted_iota(jnp.int32, (128, NX), 1)
    dn = (((0,), (0,)), ((), ()))

    NYT = NY // RB
    NSTEP = B * NYT
    si = bb * NYT + yt
    par = si % 2

    def metas_for(s):
        sb = s // NYT
        syt = s - sb * NYT
        g0 = sb * NY + syt * RB
        out = []
        for rr in range(RB):
            midx = midx_sm[g0 + rr]
            out.append((pl.multiple_of(starts_sm[midx], 128), cnts_sm[midx]))
        return out

    def blk_cps(p, bA, bB):
        return [
            pltpu.make_async_copy(feat.at[pl.ds(bA, RB), pl.ds(0, 32), :],
                                  fbufA.at[p], sems_ab.at[p, 0]),
            pltpu.make_async_copy(feat.at[pl.ds(bB, RB), pl.ds(0, 32), :],
                                  fbufB.at[p], sems_ab.at[p, 1]),
            pltpu.make_async_copy(slot_c.at[pl.ds(bA, RB), :],
                                  sloA.at[p], sems_ab.at[p, 2]),
            pltpu.make_async_copy(slot_c.at[pl.ds(bB, RB), :],
                                  sloB.at[p], sems_ab.at[p, 3]),
        ]

    def xchunk_cp(p, rr, blk, cc):
        return pltpu.make_async_copy(
            feat.at[blk, pl.ds(32 * cc, 32), :],
            fbufX.at[p, rr, cc - 1], sems_x.at[p, rr, cc - 1])

    def fire_all(s, p):
        ms = metas_for(s)
        bA = ms[0][0] // 128
        bB = ms[RB - 1][0] // 128 - (RB - 1)
        for cp in blk_cps(p, bA, bB):
            cp.start()
        for rr in range(RB):
            start, cnt = ms[rr]
            fast = jnp.logical_and(
                jnp.logical_or(start == (bA + rr) * 128,
                               start == (bB + rr) * 128), cnt <= 128)
            for cc in range(1, 4):
                @pl.when(jnp.logical_and(fast, cnt > 32 * cc))
                def _(p=p, rr=rr, start=start, cc=cc):
                    xchunk_cp(p, rr, start // 128, cc).start()

    def chunk_acc(f, sm_t, cc):
        sub = lax.slice(sm_t, (32 * cc, 0), (32 * cc + 32, 1))
        oh = (sub == iota_t).astype(jnp.bfloat16)
        return lax.dot_general(f[:, :C], oh, dn,
                               preferred_element_type=jnp.float32)

    @pl.when(si == 0)
    def _():
        fire_all(si, par)

    @pl.when(si + 1 < NSTEP)
    def _():
        fire_all(si + 1, 1 - par)

    metas = metas_for(si)
    bA = metas[0][0] // 128
    bB = metas[RB - 1][0] // 128 - (RB - 1)
    for cp in blk_cps(par, bA, bB):
        cp.wait()

    for rr in range(RB):
        start, cnt = metas[rr]
        inA = start == (bA + rr) * 128
        inB = start == (bB + rr) * 128
        fast = jnp.logical_and(jnp.logical_or(inA, inB), cnt <= 128)

        @pl.when(fast)
        def _(rr=rr, start=start, cnt=cnt, inA=inA):
            svec = jnp.where(inA, sloA[par, rr], sloB[par, rr]) & 511
            sm = jnp.where(lax.iota(jnp.int32, 128) < cnt, svec, 4096)
            sm_t = jnp.transpose(sm.reshape(1, 128), (1, 0))
            f0 = jnp.where(inA, fbufA[par, rr], fbufB[par, rr])
            o_ref[0, :, rr, :] = chunk_acc(f0, sm_t, 0)
            for cc in range(1, 4):
                @pl.when(cnt > 32 * cc)
                def _(cc=cc, rr=rr, sm_t=sm_t):
                    xchunk_cp(par, rr, start // 128, cc).wait()
                    o_ref[0, :, rr, :] += chunk_acc(fbufX[par, rr, cc - 1],
                                                    sm_t, cc)

        @pl.when(jnp.logical_not(fast))
        def _(rr=rr, start=start, cnt=cnt):
            o_ref[0, :, rr, :] = jnp.zeros((C, NX), jnp.float32)

            def big(c, carry):
                pltpu.make_async_copy(feat.at[start // 128 + c], xbuf,
                                      sem_x1).start()
                pltpu.make_async_copy(feat.at[start // 128 + c], xbuf,
                                      sem_x1).wait()
                pltpu.make_async_copy(slot_c.at[start // 128 + c], xslo,
                                      sem_x1).start()
                pltpu.make_async_copy(slot_c.at[start // 128 + c], xslo,
                                      sem_x1).wait()
                f = xbuf[...]
                ent = lax.iota(jnp.int32, 128) + 128 * c
                smx = jnp.where(ent < cnt, xslo[...] & 511, 4096)
                smx_t = jnp.transpose(smx.reshape(1, 128), (1, 0))
                oh = (smx_t == iota_t128).astype(jnp.bfloat16)
                o_ref[0, :, rr, :] += lax.dot_general(
                    f[:, :C], oh, dn, preferred_element_type=jnp.float32)
                return carry
            lax.fori_loop(0, (cnt + 127) // 128, big, 0)


def _tc_stage(feat, slot_c, starts, cnts):
    grid_spec = pltpu.PrefetchScalarGridSpec(
        num_scalar_prefetch=3,
        grid=(B, NY // RB),
        in_specs=[
            pl.BlockSpec(memory_space=pltpu.MemorySpace.HBM),
            pl.BlockSpec(memory_space=pltpu.MemorySpace.HBM),
        ],
        out_specs=pl.BlockSpec((1, C, RB, NX),
                               lambda b, y, s_r, c_r, m_r: (b, 0, y, 0)),
        scratch_shapes=[
            pltpu.VMEM((2, RB, 32, 128), jnp.float32),
            pltpu.VMEM((2, RB, 32, 128), jnp.float32),
            pltpu.VMEM((2, RB, 128), jnp.int32),
            pltpu.VMEM((2, RB, 128), jnp.int32),
            pltpu.VMEM((2, RB, 3, 32, 128), jnp.float32),
            pltpu.VMEM((128, 128), jnp.float32),
            pltpu.VMEM((128,), jnp.int32),
            pltpu.SemaphoreType.DMA((2, 4)),
            pltpu.SemaphoreType.DMA((2, RB, 3)),
            pltpu.SemaphoreType.DMA,
        ],
    )
    midx_map = ((jnp.arange(B * NY, dtype=jnp.int32) // ROWS_PER_TILE) * 64
                + jnp.arange(B * NY, dtype=jnp.int32) % ROWS_PER_TILE)
    return pl.pallas_call(
        _tc_body,
        grid_spec=grid_spec,
        out_shape=jax.ShapeDtypeStruct((B, C, NY, NX), jnp.float32),
    )(starts, cnts, midx_map, feat, slot_c)


def kernel(voxel_features, coords, batch_size, output_shape):
    c0 = coords[:, 0]
    key = c0 * CANVAS + coords[:, 2] * NX + coords[:, 3]
    key = jnp.where(c0 < batch_size, key, S_TOT).astype(jnp.int32)
    vf2 = voxel_features.reshape(P // 2, 2 * C)
    feat, slot_c, starts, cnts = _sc_stage(vf2, key)
    return _tc_stage(feat, slot_c, starts, cnts)
